# baseline (speedup 1.0000x reference)
import jax
import jax.numpy as jnp
from jax import lax
from jax.experimental import pallas as pl
from jax.experimental.pallas import tpu as pltpu
from jax.experimental.pallas import tpu_sc as plsc

SF = 2
B, C, T = 64, 256, 4096
R = 4
RT = R * T
NBLK = T // 16
NC, NS = 2, 16
SAMPLES_PER_W = B // (NC * NS)
N_CHUNK = SAMPLES_PER_W * (C // R)
CPS = C // R


def _compute_chunk(ci, b0, in_v, out_v, k_v, iota):
    bi = lax.div(ci, CPS)
    k_vec = plsc.load_gather(k_v, [jnp.full((16,), b0 + bi, jnp.int32)]) - SF
    base = iota - k_vec
    for r in range(R):
        row0 = r * T
        colL = jnp.where(base < 0, -base, base)
        out_v[pl.ds(row0, 16)] = plsc.load_gather(in_v, [colL + row0])

        def blk(j, idx, row0=row0):
            out_v[pl.ds(row0 + j * 16, 16)] = plsc.load_gather(in_v, [idx])
            return idx + 16

        lax.fori_loop(1, NBLK - 1, blk, base + row0 + 16, unroll=8)

        colR = base + (NBLK - 1) * 16
        colR = jnp.where(colR > T - 1, 2 * (T - 1) - colR, colR)
        out_v[pl.ds(row0 + (NBLK - 1) * 16, 16)] = plsc.load_gather(
            in_v, [colR + row0])


def _body(x_hbm, k_hbm, out_hbm, k_v, in0, in1, out0, out1,
          sin0, sin1, sout0, sout1):
    wid = lax.axis_index("s") * NC + lax.axis_index("c")
    b0 = wid * SAMPLES_PER_W
    base0 = b0 * (C * T)
    pltpu.sync_copy(k_hbm, k_v)
    iota = lax.iota(jnp.int32, 16)

    def start_in(ci, buf, sem):
        pltpu.make_async_copy(
            x_hbm.at[pl.ds(base0 + ci * RT, RT)], buf, sem).start()

    def wait_in(buf, sem):
        pltpu.make_async_copy(x_hbm.at[pl.ds(base0, RT)], buf, sem).wait()

    def start_out(ci, buf, sem):
        pltpu.make_async_copy(
            buf, out_hbm.at[pl.ds(base0 + ci * RT, RT)], sem).start()

    def wait_out(buf, sem):
        pltpu.make_async_copy(buf, out_hbm.at[pl.ds(base0, RT)], sem).wait()

    def compute(ci, in_v, out_v):
        _compute_chunk(ci, b0, in_v, out_v, k_v, iota)

    bufs = ((in0, sin0, out0, sout0), (in1, sin1, out1, sout1))

    start_in(0, in0, sin0)
    start_in(1, in1, sin1)
    for p in range(2):
        iv, isem, ov, osem = bufs[p]
        wait_in(iv, isem)
        compute(jnp.int32(p), iv, ov)
        start_out(p, ov, osem)
        start_in(p + 2, iv, isem)

    def steady(g, carry):
        for p in range(2):
            iv, isem, ov, osem = bufs[p]
            ci = 2 * g + p
            wait_in(iv, isem)
            wait_out(ov, osem)
            compute(ci, iv, ov)
            start_out(ci, ov, osem)
            start_in(ci + 2, iv, isem)
        return carry

    lax.fori_loop(1, N_CHUNK // 2 - 1, steady, 0)

    for p in range(2):
        iv, isem, ov, osem = bufs[p]
        ci = N_CHUNK - 2 + p
        wait_in(iv, isem)
        wait_out(ov, osem)
        compute(jnp.int32(ci), iv, ov)
        start_out(ci, ov, osem)
    for p in range(2):
        _, _, ov, osem = bufs[p]
        wait_out(ov, osem)


@jax.jit
def kernel(x, k_list):
    mesh = plsc.VectorSubcoreMesh(core_axis_name="c", subcore_axis_name="s")
    run = pl.kernel(
        _body,
        out_type=jax.ShapeDtypeStruct((B * C * T,), jnp.float32),
        mesh=mesh,
        scratch_types=[
            pltpu.VMEM((B,), jnp.int32),
            pltpu.VMEM((RT,), jnp.float32),
            pltpu.VMEM((RT,), jnp.float32),
            pltpu.VMEM((RT,), jnp.float32),
            pltpu.VMEM((RT,), jnp.float32),
            pltpu.SemaphoreType.DMA,
            pltpu.SemaphoreType.DMA,
            pltpu.SemaphoreType.DMA,
            pltpu.SemaphoreType.DMA,
        ],
        compiler_params=pltpu.CompilerParams(needs_layout_passes=False),
    )
    out = run(x.reshape(B * C * T), k_list.astype(jnp.int32))
    return out.reshape(B, C, T)


# speedup vs baseline: 4.8518x; 4.8518x over previous
import jax
import jax.numpy as jnp
from jax import lax
from jax.experimental import pallas as pl
from jax.experimental.pallas import tpu as pltpu
from jax.experimental.pallas import tpu_sc as plsc

SF = 2
B, C, T = 64, 256, 4096
R = 4
RT = R * T
NBLK = T // 16
NC, NS = 2, 16
SAMPLES_PER_W = B // (NC * NS)
CPS = C // R
LOG_CPS = 6
N_CHUNK = SAMPLES_PER_W * CPS


def _compute_chunk(ci, b0, in_v, out_v, k_v, iota):
    bi = b0 + lax.shift_right_logical(ci, LOG_CPS)
    k_vec = plsc.load_gather(k_v, [jnp.full((16,), bi, jnp.int32)]) - SF
    base = iota - k_vec
    for r in range(R):
        row0 = r * T
        colL = jnp.where(base < 0, -base, base)
        out_v[pl.ds(row0, 16)] = plsc.load_gather(in_v, [colL + row0])

        @plsc.parallel_loop(1, NBLK - 1, unroll=8, carry=base + row0 + 16)
        def blk(j, idx, row0=row0):
            out_v[pl.ds(row0 + j * 16, 16)] = plsc.load_gather(in_v, [idx])
            return idx + 16

        colR = base + (NBLK - 1) * 16
        colR = jnp.where(colR > T - 1, 2 * (T - 1) - colR, colR)
        out_v[pl.ds(row0 + (NBLK - 1) * 16, 16)] = plsc.load_gather(
            in_v, [colR + row0])


def _body(x_hbm, k_hbm, out_hbm, k_v, in0, in1, out0, out1,
          sin0, sin1, sout0, sout1):
    wid = lax.axis_index("s") * NC + lax.axis_index("c")
    b0 = wid * SAMPLES_PER_W
    pltpu.sync_copy(k_hbm, k_v)
    iota = lax.iota(jnp.int32, 16)

    def src_at(ci):
        b = b0 + lax.shift_right_logical(ci, LOG_CPS)
        c0 = lax.shift_left(ci & (CPS - 1), 2)
        return b, c0

    def start_in(ci, buf, sem):
        b, c0 = src_at(ci)
        for r in range(R):
            pltpu.make_async_copy(
                x_hbm.at[b, c0 + r, :], buf.at[pl.ds(r * T, T)], sem).start()

    def wait_in(buf, sem):
        for r in range(R):
            pltpu.make_async_copy(
                x_hbm.at[0, 0, :], buf.at[pl.ds(r * T, T)], sem).wait()

    def start_out(ci, buf, sem):
        b, c0 = src_at(ci)
        for r in range(R):
            pltpu.make_async_copy(
                buf.at[pl.ds(r * T, T)], out_hbm.at[b, c0 + r, :], sem).start()

    def wait_out(buf, sem):
        for r in range(R):
            pltpu.make_async_copy(
                buf.at[pl.ds(r * T, T)], out_hbm.at[0, 0, :], sem).wait()

    def compute(ci, in_v, out_v):
        _compute_chunk(ci, b0, in_v, out_v, k_v, iota)

    bufs = ((in0, sin0, out0, sout0), (in1, sin1, out1, sout1))

    start_in(jnp.int32(0), in0, sin0)
    start_in(jnp.int32(1), in1, sin1)
    for p in range(2):
        iv, isem, ov, osem = bufs[p]
        wait_in(iv, isem)
        compute(jnp.int32(p), iv, ov)
        start_out(jnp.int32(p), ov, osem)
        start_in(jnp.int32(p + 2), iv, isem)

    def steady(g, carry):
        for p in range(2):
            iv, isem, ov, osem = bufs[p]
            ci = 2 * g + p
            wait_in(iv, isem)
            wait_out(ov, osem)
            compute(ci, iv, ov)
            start_out(ci, ov, osem)
            start_in(ci + 2, iv, isem)
        return carry

    lax.fori_loop(1, N_CHUNK // 2 - 1, steady, 0)

    for p in range(2):
        iv, isem, ov, osem = bufs[p]
        ci = jnp.int32(N_CHUNK - 2 + p)
        wait_in(iv, isem)
        wait_out(ov, osem)
        compute(ci, iv, ov)
        start_out(ci, ov, osem)
    for p in range(2):
        _, _, ov, osem = bufs[p]
        wait_out(ov, osem)


@jax.jit
def kernel(x, k_list):
    mesh = plsc.VectorSubcoreMesh(core_axis_name="c", subcore_axis_name="s")
    run = pl.kernel(
        _body,
        out_type=jax.ShapeDtypeStruct((B, C, T), jnp.float32),
        mesh=mesh,
        scratch_types=[
            pltpu.VMEM((B,), jnp.int32),
            pltpu.VMEM((RT,), jnp.float32),
            pltpu.VMEM((RT,), jnp.float32),
            pltpu.VMEM((RT,), jnp.float32),
            pltpu.VMEM((RT,), jnp.float32),
            pltpu.SemaphoreType.DMA,
            pltpu.SemaphoreType.DMA,
            pltpu.SemaphoreType.DMA,
            pltpu.SemaphoreType.DMA,
        ],
        compiler_params=pltpu.CompilerParams(needs_layout_passes=False),
    )
    return run(x, k_list.astype(jnp.int32))
